# baseline (device time: 14949 ns/iter reference)
import jax
import jax.numpy as jnp
from jax import lax
from jax.experimental import pallas as pl
from jax.experimental.pallas import tpu as pltpu

N_DEV = 4


def kernel(Q, K, V):
    b, sq, h, d = Q.shape
    skv = K.shape[1]
    scale = d ** -0.5

    q2 = Q.reshape(b, h, d)
    kt = jnp.transpose(K, (0, 2, 3, 1))
    vt = jnp.transpose(V, (0, 2, 3, 1))

    def body(q_ref, k_ref, v_ref, out_ref,
             comm, kbuf, vbuf, load_sems, send_sems, recv_sems):
        my = lax.axis_index("i")

        loads = []
        for bb in range(b):
            ck = pltpu.make_async_copy(
                k_ref.at[bb], kbuf.at[bb], load_sems.at[bb])
            cv = pltpu.make_async_copy(
                v_ref.at[bb], vbuf.at[bb], load_sems.at[b + bb])
            ck.start()
            cv.start()
            loads.append((ck, cv))

        barrier_sem = pltpu.get_barrier_semaphore()
        for off in (1, 2, 3):
            nbr = lax.rem(my + off, N_DEV)
            pl.semaphore_signal(
                barrier_sem, inc=1,
                device_id=(nbr,), device_id_type=pl.DeviceIdType.MESH,
            )
        pl.semaphore_wait(barrier_sem, N_DEV - 1)

        rdmas = []
        for bb in range(b):
            ck, cv = loads[bb]
            ck.wait()
            cv.wait()
            qb = q_ref[bb].astype(jnp.bfloat16)
            kb = kbuf[bb].astype(jnp.bfloat16)
            vb = vbuf[bb].astype(jnp.bfloat16)
            s = lax.dot_general(
                qb, kb, (((1,), (1,)), ((0,), (0,))),
                preferred_element_type=jnp.float32) * scale
            m = jnp.max(s, axis=-1, keepdims=True)
            p = jnp.exp(s - m)
            l = jnp.sum(p, axis=-1, keepdims=True)
            u = lax.dot_general(
                p.astype(jnp.bfloat16), vb, (((1,), (2,)), ((0,), (0,))),
                preferred_element_type=jnp.float32)
            comm[0, bb, :, 0:d] = u
            comm[0, bb, :, d:d + 1] = m
            comm[0, bb, :, d + 1:d + 2] = l

            row = []
            for off in (2, 1, 3):
                dst = lax.rem(my + off, N_DEV)
                slot = N_DEV - off
                r = pltpu.make_async_remote_copy(
                    src_ref=comm.at[0, bb],
                    dst_ref=comm.at[slot, bb],
                    send_sem=send_sems.at[bb, off - 1],
                    recv_sem=recv_sems.at[bb, slot - 1],
                    device_id=(dst,),
                    device_id_type=pl.DeviceIdType.MESH,
                )
                r.start()
                row.append(r)
            rdmas.append(row)

        for bb in range(b):
            for r in rdmas[bb]:
                r.wait_recv()
            c = comm[:, bb]
            u4 = c[:, :, 0:d]
            m4 = c[:, :, d:d + 1]
            l4 = c[:, :, d + 1:d + 2]
            mx = jnp.max(m4, axis=0, keepdims=True)
            w = jnp.exp(m4 - mx)
            lsum = jnp.sum(l4 * w, axis=0)
            usum = jnp.sum(u4 * w, axis=0)
            out_ref[bb] = (usum / lsum).reshape(sq, h, d)

        for row in rdmas:
            for r in row:
                r.wait_send()

    return pl.pallas_call(
        body,
        out_shape=jax.ShapeDtypeStruct((b, sq, h, d), jnp.float32),
        in_specs=[
            pl.BlockSpec(memory_space=pltpu.VMEM),
            pl.BlockSpec(memory_space=pltpu.MemorySpace.HBM),
            pl.BlockSpec(memory_space=pltpu.MemorySpace.HBM),
        ],
        out_specs=pl.BlockSpec(memory_space=pltpu.VMEM),
        scratch_shapes=[
            pltpu.VMEM((N_DEV, b, h, d + 2), jnp.float32),
            pltpu.VMEM((b, h, d, skv), jnp.float32),
            pltpu.VMEM((b, h, d, skv), jnp.float32),
            pltpu.SemaphoreType.DMA((2 * b,)),
            pltpu.SemaphoreType.DMA((b, 3)),
            pltpu.SemaphoreType.DMA((b, 3)),
        ],
        compiler_params=pltpu.CompilerParams(
            collective_id=0,
            vmem_limit_bytes=100 * 1024 * 1024,
        ),
    )(q2, kt, vt)


# device time: 14604 ns/iter; 1.0236x vs baseline; 1.0236x over previous
import jax
import jax.numpy as jnp
from jax import lax
from jax.experimental import pallas as pl
from jax.experimental.pallas import tpu as pltpu

N_DEV = 4


def kernel(Q, K, V):
    b, sq, h, d = Q.shape
    skv = K.shape[1]
    scale = d ** -0.5

    q2 = Q.reshape(b, h, d)
    kt = jnp.transpose(K, (0, 2, 3, 1))
    vt = jnp.transpose(V, (0, 2, 3, 1))

    def body(q_ref, k_ref, v_ref, out_ref,
             comm, kbuf, vbuf, load_sems, send_sems, recv_sems):
        my = lax.axis_index("i")

        loads = []
        for bb in range(b):
            ck = pltpu.make_async_copy(
                k_ref.at[bb], kbuf.at[bb], load_sems.at[bb])
            cv = pltpu.make_async_copy(
                v_ref.at[bb], vbuf.at[bb], load_sems.at[b + bb])
            ck.start()
            cv.start()
            loads.append((ck, cv))

        barrier_sem = pltpu.get_barrier_semaphore()
        for off in (1, 2, 3):
            nbr = lax.rem(my + off, N_DEV)
            pl.semaphore_signal(
                barrier_sem, inc=1,
                device_id=(nbr,), device_id_type=pl.DeviceIdType.MESH,
            )
        pl.semaphore_wait(barrier_sem, N_DEV - 1)

        rdmas = []
        for bb in range(b):
            ck, cv = loads[bb]
            ck.wait()
            cv.wait()
            qb = q_ref[bb].astype(jnp.bfloat16)
            kb = kbuf[bb].astype(jnp.bfloat16)
            vb = vbuf[bb].astype(jnp.bfloat16)
            s = lax.dot_general(
                qb, kb, (((1,), (1,)), ((0,), (0,))),
                preferred_element_type=jnp.float32) * scale
            m = jnp.max(s, axis=-1, keepdims=True)
            p = jnp.exp(s - m)
            l = jnp.sum(p, axis=-1, keepdims=True)
            u = lax.dot_general(
                p.astype(jnp.bfloat16), vb, (((1,), (2,)), ((0,), (0,))),
                preferred_element_type=jnp.float32)
            comm[0, bb, :, 0:d] = u
            comm[0, bb, :, d:d + 1] = m
            comm[0, bb, :, d + 1:d + 2] = l

            if bb % 2 == 1:
                pair = bb // 2
                row = []
                for off in (2, 1, 3):
                    dst = lax.rem(my + off, N_DEV)
                    slot = N_DEV - off
                    r = pltpu.make_async_remote_copy(
                        src_ref=comm.at[0, pl.ds(bb - 1, 2)],
                        dst_ref=comm.at[slot, pl.ds(bb - 1, 2)],
                        send_sem=send_sems.at[pair, off - 1],
                        recv_sem=recv_sems.at[pair, slot - 1],
                        device_id=(dst,),
                        device_id_type=pl.DeviceIdType.MESH,
                    )
                    r.start()
                    row.append(r)
                rdmas.append(row)

        for half in range(2):
            for pair in (2 * half, 2 * half + 1):
                for r in rdmas[pair]:
                    r.wait_recv()
            rows = pl.ds(4 * half, 4)
            c = comm[:, rows]
            u4 = c[:, :, :, 0:d]
            m4 = c[:, :, :, d:d + 1]
            l4 = c[:, :, :, d + 1:d + 2]
            mx = jnp.max(m4, axis=0, keepdims=True)
            w = jnp.exp(m4 - mx)
            lsum = jnp.sum(l4 * w, axis=0)
            usum = jnp.sum(u4 * w, axis=0)
            out_ref[rows] = (usum / lsum).reshape(4, sq, h, d)

        for row in rdmas:
            for r in row:
                r.wait_send()

    return pl.pallas_call(
        body,
        out_shape=jax.ShapeDtypeStruct((b, sq, h, d), jnp.float32),
        in_specs=[
            pl.BlockSpec(memory_space=pltpu.VMEM),
            pl.BlockSpec(memory_space=pltpu.MemorySpace.HBM),
            pl.BlockSpec(memory_space=pltpu.MemorySpace.HBM),
        ],
        out_specs=pl.BlockSpec(memory_space=pltpu.VMEM),
        scratch_shapes=[
            pltpu.VMEM((N_DEV, b, h, d + 2), jnp.float32),
            pltpu.VMEM((b, h, d, skv), jnp.float32),
            pltpu.VMEM((b, h, d, skv), jnp.float32),
            pltpu.SemaphoreType.DMA((2 * b,)),
            pltpu.SemaphoreType.DMA((b // 2, 3)),
            pltpu.SemaphoreType.DMA((b // 2, 3)),
        ],
        compiler_params=pltpu.CompilerParams(
            collective_id=0,
            vmem_limit_bytes=100 * 1024 * 1024,
        ),
    )(q2, kt, vt)
